# baseline (speedup 1.0000x reference)
import functools

import jax
import jax.numpy as jnp
from jax import lax
from jax.experimental import pallas as pl
from jax.experimental.pallas import tpu as pltpu
from jax.experimental.pallas import tpu_sc as plsc

_B = 16384
_V = 100000
_D = 10
_EPS = 1e-5

_NC, _NS = 2, 16
_NW = _NC * _NS
_BPW = _B // _NW
_EPW = _BPW * _D
_CHUNK = 128
_L = 16

_mesh = plsc.VectorSubcoreMesh(core_axis_name="c", subcore_axis_name="s")


@functools.partial(
    pl.kernel,
    out_type=jax.ShapeDtypeStruct((_D, _B), jnp.float32),
    mesh=_mesh,
    scratch_types=[
        pltpu.VMEM((_BPW,), jnp.int32),
        pltpu.VMEM((_EPW,), jnp.int32),
        pltpu.VMEM((_D, _BPW), jnp.float32),
        pltpu.SemaphoreType.DMA,
    ],
    compiler_params=pltpu.CompilerParams(
        use_tc_tiling_on_sc=False, needs_layout_passes=False),
)
def _sc_gather(tbl, idx, out, iv, ev, row, sem):
    wid = lax.axis_index("s") * _NC + lax.axis_index("c")
    row0 = wid * _BPW
    pltpu.sync_copy(idx.at[pl.ds(row0, _BPW)], iv)
    for j in range(_BPW // _L):
        v = iv[pl.ds(j * _L, _L)]
        for k in range(_D):
            ev[pl.ds(k * _BPW + j * _L, _L)] = v + (k * _V)
    copies = []
    for k in range(_D):
        isl = pl.ds(k * _BPW, _BPW)
        copies.append(pltpu.async_copy(
            tbl.at[ev.at[isl]], row.at[k], sem))
    for cp in copies:
        cp.wait()
    pltpu.sync_copy(row, out.at[:, pl.ds(row0, _BPW)])


@functools.partial(
    pl.kernel,
    out_type=(jax.ShapeDtypeStruct((_D, _B), jnp.float32),) * 2,
    mesh=_mesh,
    scratch_types=[
        pltpu.VMEM((_BPW,), jnp.int32),
        pltpu.VMEM((_BPW,), jnp.int32),
        pltpu.VMEM((_EPW,), jnp.int32),
        pltpu.VMEM((_EPW,), jnp.int32),
        pltpu.VMEM((_D, _BPW), jnp.float32),
        pltpu.VMEM((_D, _BPW), jnp.float32),
        pltpu.SemaphoreType.DMA,
    ],
    compiler_params=pltpu.CompilerParams(
        use_tc_tiling_on_sc=False, needs_layout_passes=False),
)
def _sc_gather2(f_tbl, o_tbl, f_idx, o_idx, out_f, out_o,
                fiv, oiv, fe, oe, frow, orow, sem):
    wid = lax.axis_index("s") * _NC + lax.axis_index("c")
    row0 = wid * _BPW
    pltpu.sync_copy(f_idx.at[pl.ds(row0, _BPW)], fiv)
    pltpu.sync_copy(o_idx.at[pl.ds(row0, _BPW)], oiv)
    for iv, ev in ((fiv, fe), (oiv, oe)):
        for j in range(_BPW // _L):
            v = iv[pl.ds(j * _L, _L)]
            for k in range(_D):
                ev[pl.ds(k * _BPW + j * _L, _L)] = v + (k * _V)
    copies = []
    for tbl, ev, row in ((f_tbl, fe, frow), (o_tbl, oe, orow)):
        for k in range(_D):
            isl = pl.ds(k * _BPW, _BPW)
            copies.append(pltpu.async_copy(
                tbl.at[ev.at[isl]], row.at[k], sem))
    for cp in copies:
        cp.wait()
    pltpu.sync_copy(frow, out_f.at[:, pl.ds(row0, _BPW)])
    pltpu.sync_copy(orow, out_o.at[:, pl.ds(row0, _BPW)])


_BLK = 1024


def _mlp_body(xc, xf, xo, xr, a_ref, b_ref, c_ref, x_ref,
              b1_ref, w2_ref, b2_ref, w3_ref, b3_ref, out):
    h = (jnp.dot(a_ref[...], xc[...], preferred_element_type=jnp.float32)
         + jnp.dot(b_ref[...], xf[...], preferred_element_type=jnp.float32)
         + jnp.dot(c_ref[...], xo[...], preferred_element_type=jnp.float32)
         + jnp.dot(x_ref[...], xr[...], preferred_element_type=jnp.float32)
         + b1_ref[...])
    h = jnp.maximum(h, 0.0)
    h = jnp.maximum(
        jnp.dot(w2_ref[...], h, preferred_element_type=jnp.float32)
        + b2_ref[...], 0.0)
    out[...] = (jnp.dot(w3_ref[...], h, preferred_element_type=jnp.float32)
                + b3_ref[...])


def _full(shape):
    return pl.BlockSpec(shape, lambda i: (0, 0))


_mlp_call = pl.pallas_call(
    _mlp_body,
    grid=(_B // _BLK,),
    in_specs=[
        pl.BlockSpec((_D, _BLK), lambda i: (0, i)),
        pl.BlockSpec((_D, _BLK), lambda i: (0, i)),
        pl.BlockSpec((_D, _BLK), lambda i: (0, i)),
        pl.BlockSpec((9, _BLK), lambda i: (0, i)),
        _full((32, _D)),
        _full((32, _D)),
        _full((32, _D)),
        _full((32, 9)),
        _full((32, 1)),
        _full((16, 32)),
        _full((16, 1)),
        _full((3, 16)),
        _full((3, 1)),
    ],
    out_specs=pl.BlockSpec((3, _BLK), lambda i: (0, i)),
    out_shape=jax.ShapeDtypeStruct((3, _B), jnp.float32),
)


def kernel(x, country_table, coffee_table, occupation_table,
           bn_weight, bn_bias, W1, b1, W2, b2, W3, b3):
    c_idx = x[:, 1].astype(jnp.int32)
    f_idx = x[:, 2].astype(jnp.int32)
    o_idx = x[:, 8].astype(jnp.int32)

    gc = _sc_gather(country_table.T.reshape(-1), c_idx)
    gf, go = _sc_gather2(coffee_table.T.reshape(-1),
                         occupation_table.T.reshape(-1), f_idx, o_idx)

    s = bn_weight * (1.0 / jnp.sqrt(1.0 + _EPS))
    w1s = W1 * s[:, None]
    b1p = (bn_bias @ W1 + b1).reshape(32, 1)
    wc_t = w1s[0:10].T
    wf_t = w1s[10:20].T
    wo_t = w1s[20:30].T
    wx_t = jnp.zeros((9, 32), jnp.float32).at[
        jnp.array([0, 3, 4, 5, 6, 7], dtype=jnp.int32)].set(w1s[30:36]).T

    out_t = _mlp_call(gc, gf, go, x.T,
                      wc_t, wf_t, wo_t, wx_t,
                      b1p, W2.T, b2.reshape(16, 1), W3.T, b3.reshape(3, 1))
    return out_t.T


# speedup vs baseline: 1.0694x; 1.0694x over previous
import functools

import jax
import jax.numpy as jnp
from jax import lax
from jax.experimental import pallas as pl
from jax.experimental.pallas import tpu as pltpu
from jax.experimental.pallas import tpu_sc as plsc

_B = 16384
_V = 100000
_D = 10
_EPS = 1e-5

_NC, _NS = 2, 16
_NW = _NC * _NS
_BPW = _B // _NW
_EPW = _BPW * _D
_CHUNK = 128
_L = 16

_mesh = plsc.VectorSubcoreMesh(core_axis_name="c", subcore_axis_name="s")


@functools.partial(
    pl.kernel,
    out_type=jax.ShapeDtypeStruct((_D, _B), jnp.float32),
    mesh=_mesh,
    scratch_types=[
        pltpu.VMEM((_BPW,), jnp.int32),
        pltpu.VMEM((_EPW,), jnp.int32),
        pltpu.VMEM((_D, _BPW), jnp.float32),
        pltpu.SemaphoreType.DMA,
    ],
    compiler_params=pltpu.CompilerParams(
        use_tc_tiling_on_sc=False, needs_layout_passes=False),
)
def _sc_gather(tbl, idx, out, iv, ev, row, sem):
    wid = lax.axis_index("s") * _NC + lax.axis_index("c")
    row0 = wid * _BPW
    pltpu.sync_copy(idx.at[pl.ds(row0, _BPW)], iv)
    for j in range(_BPW // _L):
        v = iv[pl.ds(j * _L, _L)]
        for k in range(_D):
            ev[pl.ds(k * _BPW + j * _L, _L)] = v + (k * _V)
    copies = []
    for k in range(_D):
        isl = pl.ds(k * _BPW, _BPW)
        copies.append(pltpu.async_copy(
            tbl.at[ev.at[isl]], row.at[k], sem))
    for cp in copies:
        cp.wait()
    pltpu.sync_copy(row, out.at[:, pl.ds(row0, _BPW)])


@functools.partial(
    pl.kernel,
    out_type=(jax.ShapeDtypeStruct((_D, _B), jnp.float32),) * 2,
    mesh=_mesh,
    scratch_types=[
        pltpu.VMEM((_BPW,), jnp.int32),
        pltpu.VMEM((_BPW,), jnp.int32),
        pltpu.VMEM((_EPW,), jnp.int32),
        pltpu.VMEM((_EPW,), jnp.int32),
        pltpu.VMEM((_D, _BPW), jnp.float32),
        pltpu.VMEM((_D, _BPW), jnp.float32),
        pltpu.SemaphoreType.DMA,
    ],
    compiler_params=pltpu.CompilerParams(
        use_tc_tiling_on_sc=False, needs_layout_passes=False),
)
def _sc_gather2(f_tbl, o_tbl, f_idx, o_idx, out_f, out_o,
                fiv, oiv, fe, oe, frow, orow, sem):
    wid = lax.axis_index("s") * _NC + lax.axis_index("c")
    row0 = wid * _BPW
    pltpu.sync_copy(f_idx.at[pl.ds(row0, _BPW)], fiv)
    pltpu.sync_copy(o_idx.at[pl.ds(row0, _BPW)], oiv)
    for iv, ev in ((fiv, fe), (oiv, oe)):
        for j in range(_BPW // _L):
            v = iv[pl.ds(j * _L, _L)]
            for k in range(_D):
                ev[pl.ds(k * _BPW + j * _L, _L)] = v + (k * _V)
    copies = []
    for tbl, ev, row in ((f_tbl, fe, frow), (o_tbl, oe, orow)):
        for k in range(_D):
            isl = pl.ds(k * _BPW, _BPW)
            copies.append(pltpu.async_copy(
                tbl.at[ev.at[isl]], row.at[k], sem))
    for cp in copies:
        cp.wait()
    pltpu.sync_copy(frow, out_f.at[:, pl.ds(row0, _BPW)])
    pltpu.sync_copy(orow, out_o.at[:, pl.ds(row0, _BPW)])


_BLK = 2048


def _mlp_body(xc, xf, xo, xr, a_ref, b_ref, c_ref, x_ref,
              b1_ref, w2_ref, b2_ref, w3_ref, b3_ref, out):
    h = (jnp.dot(a_ref[...], xc[...], preferred_element_type=jnp.float32)
         + jnp.dot(b_ref[...], xf[...], preferred_element_type=jnp.float32)
         + jnp.dot(c_ref[...], xo[...], preferred_element_type=jnp.float32)
         + jnp.dot(x_ref[...], xr[...], preferred_element_type=jnp.float32)
         + b1_ref[...])
    h = jnp.maximum(h, 0.0)
    h = jnp.maximum(
        jnp.dot(w2_ref[...], h, preferred_element_type=jnp.float32)
        + b2_ref[...], 0.0)
    out[...] = (jnp.dot(w3_ref[...], h, preferred_element_type=jnp.float32)
                + b3_ref[...])


def _full(shape):
    return pl.BlockSpec(shape, lambda i: (0, 0))


_mlp_call = pl.pallas_call(
    _mlp_body,
    grid=(_B // _BLK,),
    in_specs=[
        pl.BlockSpec((_D, _BLK), lambda i: (0, i)),
        pl.BlockSpec((_D, _BLK), lambda i: (0, i)),
        pl.BlockSpec((_D, _BLK), lambda i: (0, i)),
        pl.BlockSpec((9, _BLK), lambda i: (0, i)),
        _full((32, _D)),
        _full((32, _D)),
        _full((32, _D)),
        _full((32, 9)),
        _full((32, 1)),
        _full((16, 32)),
        _full((16, 1)),
        _full((3, 16)),
        _full((3, 1)),
    ],
    out_specs=pl.BlockSpec((3, _BLK), lambda i: (0, i)),
    out_shape=jax.ShapeDtypeStruct((3, _B), jnp.float32),
)


def kernel(x, country_table, coffee_table, occupation_table,
           bn_weight, bn_bias, W1, b1, W2, b2, W3, b3):
    c_idx = x[:, 1].astype(jnp.int32)
    f_idx = x[:, 2].astype(jnp.int32)
    o_idx = x[:, 8].astype(jnp.int32)

    gc, gf = _sc_gather2(country_table.T.reshape(-1),
                         coffee_table.T.reshape(-1), c_idx, f_idx)
    go = _sc_gather(occupation_table.T.reshape(-1), o_idx)

    s = bn_weight * (1.0 / jnp.sqrt(1.0 + _EPS))
    w1s = W1 * s[:, None]
    b1p = (bn_bias @ W1 + b1).reshape(32, 1)
    wc_t = w1s[0:10].T
    wf_t = w1s[10:20].T
    wo_t = w1s[20:30].T
    wx_t = jnp.zeros((9, 32), jnp.float32).at[
        jnp.array([0, 3, 4, 5, 6, 7], dtype=jnp.int32)].set(w1s[30:36]).T

    out_t = _mlp_call(gc, gf, go, x.T,
                      wc_t, wf_t, wo_t, wx_t,
                      b1p, W2.T, b2.reshape(16, 1), W3.T, b3.reshape(3, 1))
    return out_t.T


# speedup vs baseline: 1.0836x; 1.0133x over previous
import functools

import jax
import jax.numpy as jnp
from jax import lax
from jax.experimental import pallas as pl
from jax.experimental.pallas import tpu as pltpu
from jax.experimental.pallas import tpu_sc as plsc

_B = 16384
_V = 100000
_D = 10
_EPS = 1e-5

_NC, _NS = 2, 16
_NW = _NC * _NS
_BPW = _B // _NW
_EPW = _BPW * _D
_CHUNK = 128
_L = 16

_mesh = plsc.VectorSubcoreMesh(core_axis_name="c", subcore_axis_name="s")


@functools.partial(
    pl.kernel,
    out_type=jax.ShapeDtypeStruct((_D, _B), jnp.float32),
    mesh=_mesh,
    scratch_types=[
        pltpu.VMEM((_BPW,), jnp.int32),
        pltpu.VMEM((_EPW,), jnp.int32),
        pltpu.VMEM((_D, _BPW), jnp.float32),
        pltpu.SemaphoreType.DMA,
    ],
    compiler_params=pltpu.CompilerParams(
        use_tc_tiling_on_sc=False, needs_layout_passes=False),
)
def _sc_gather(tbl, idx, out, iv, ev, row, sem):
    wid = lax.axis_index("s") * _NC + lax.axis_index("c")
    row0 = wid * _BPW
    pltpu.sync_copy(idx.at[pl.ds(row0, _BPW)], iv)
    for j in range(_BPW // _L):
        v = iv[pl.ds(j * _L, _L)]
        for k in range(_D):
            ev[pl.ds(k * _BPW + j * _L, _L)] = v + (k * _V)
    copies = []
    for k in range(_D):
        isl = pl.ds(k * _BPW, _BPW)
        copies.append(pltpu.async_copy(
            tbl.at[ev.at[isl]], row.at[k], sem))
    for cp in copies:
        cp.wait()
    pltpu.sync_copy(row, out.at[:, pl.ds(row0, _BPW)])


@functools.partial(
    pl.kernel,
    out_type=(jax.ShapeDtypeStruct((_D, _B), jnp.float32),) * 2,
    mesh=_mesh,
    scratch_types=[
        pltpu.VMEM((_BPW,), jnp.int32),
        pltpu.VMEM((_BPW,), jnp.int32),
        pltpu.VMEM((_EPW,), jnp.int32),
        pltpu.VMEM((_EPW,), jnp.int32),
        pltpu.VMEM((_D, _BPW), jnp.float32),
        pltpu.VMEM((_D, _BPW), jnp.float32),
        pltpu.SemaphoreType.DMA,
    ],
    compiler_params=pltpu.CompilerParams(
        use_tc_tiling_on_sc=False, needs_layout_passes=False),
)
def _sc_gather2(f_tbl, o_tbl, f_idx, o_idx, out_f, out_o,
                fiv, oiv, fe, oe, frow, orow, sem):
    wid = lax.axis_index("s") * _NC + lax.axis_index("c")
    row0 = wid * _BPW
    pltpu.sync_copy(f_idx.at[pl.ds(row0, _BPW)], fiv)
    pltpu.sync_copy(o_idx.at[pl.ds(row0, _BPW)], oiv)
    for iv, ev in ((fiv, fe), (oiv, oe)):
        for j in range(_BPW // _L):
            v = iv[pl.ds(j * _L, _L)]
            for k in range(_D):
                ev[pl.ds(k * _BPW + j * _L, _L)] = v + (k * _V)
    copies = []
    for tbl, ev, row in ((f_tbl, fe, frow), (o_tbl, oe, orow)):
        for k in range(_D):
            isl = pl.ds(k * _BPW, _BPW)
            copies.append(pltpu.async_copy(
                tbl.at[ev.at[isl]], row.at[k], sem))
    for cp in copies:
        cp.wait()
    pltpu.sync_copy(frow, out_f.at[:, pl.ds(row0, _BPW)])
    pltpu.sync_copy(orow, out_o.at[:, pl.ds(row0, _BPW)])


_BLK = 2048


def _mlp_body(xc, xf, xo, xr, a_ref, b_ref, c_ref, x_ref,
              b1_ref, w2_ref, b2_ref, w3_ref, b3_ref, out):
    h = (jnp.dot(a_ref[...], xc[...], preferred_element_type=jnp.float32)
         + jnp.dot(b_ref[...], xf[...], preferred_element_type=jnp.float32)
         + jnp.dot(c_ref[...], xo[...], preferred_element_type=jnp.float32)
         + jnp.dot(x_ref[...], xr[...], preferred_element_type=jnp.float32)
         + b1_ref[...])
    h = jnp.maximum(h, 0.0)
    h = jnp.maximum(
        jnp.dot(w2_ref[...], h, preferred_element_type=jnp.float32)
        + b2_ref[...], 0.0)
    out[...] = (jnp.dot(w3_ref[...], h, preferred_element_type=jnp.float32)
                + b3_ref[...])


def _full(shape):
    return pl.BlockSpec(shape, lambda i: (0, 0))


_mlp_call = pl.pallas_call(
    _mlp_body,
    grid=(_B // _BLK,),
    in_specs=[
        pl.BlockSpec((_D, _BLK), lambda i: (0, i)),
        pl.BlockSpec((_D, _BLK), lambda i: (0, i)),
        pl.BlockSpec((_D, _BLK), lambda i: (0, i)),
        pl.BlockSpec((9, _BLK), lambda i: (0, i)),
        _full((32, _D)),
        _full((32, _D)),
        _full((32, _D)),
        _full((32, 9)),
        _full((32, 1)),
        _full((16, 32)),
        _full((16, 1)),
        _full((3, 16)),
        _full((3, 1)),
    ],
    out_specs=pl.BlockSpec((3, _BLK), lambda i: (0, i)),
    out_shape=jax.ShapeDtypeStruct((3, _B), jnp.float32),
)


def kernel(x, country_table, coffee_table, occupation_table,
           bn_weight, bn_bias, W1, b1, W2, b2, W3, b3):
    c_idx = x[:, 1].astype(jnp.int32)
    f_idx = x[:, 2].astype(jnp.int32)
    o_idx = x[:, 8].astype(jnp.int32)

    o_flat = occupation_table.T.reshape(-1)
    country_table, coffee_table, o_flat = lax.optimization_barrier(
        (country_table, coffee_table, o_flat))
    go = _sc_gather(o_flat, o_idx)
    gc, gf = _sc_gather2(country_table.T.reshape(-1),
                         coffee_table.T.reshape(-1), c_idx, f_idx)

    s = bn_weight * (1.0 / jnp.sqrt(1.0 + _EPS))
    w1s = W1 * s[:, None]
    b1p = (bn_bias @ W1 + b1).reshape(32, 1)
    wc_t = w1s[0:10].T
    wf_t = w1s[10:20].T
    wo_t = w1s[20:30].T
    wx_t = jnp.zeros((9, 32), jnp.float32).at[
        jnp.array([0, 3, 4, 5, 6, 7], dtype=jnp.int32)].set(w1s[30:36]).T

    out_t = _mlp_call(gc, gf, go, x.T,
                      wc_t, wf_t, wo_t, wx_t,
                      b1p, W2.T, b2.reshape(16, 1), W3.T, b3.reshape(3, 1))
    return out_t.T


# speedup vs baseline: 1.0893x; 1.0053x over previous
import functools

import jax
import jax.numpy as jnp
from jax import lax
from jax.experimental import pallas as pl
from jax.experimental.pallas import tpu as pltpu
from jax.experimental.pallas import tpu_sc as plsc

_B = 16384
_V = 100000
_D = 10
_EPS = 1e-5

_NC, _NS = 2, 16
_NW = _NC * _NS
_BPW = _B // _NW
_EPW = _BPW * _D
_CHUNK = 128
_L = 16

_mesh = plsc.VectorSubcoreMesh(core_axis_name="c", subcore_axis_name="s")


@functools.partial(
    pl.kernel,
    out_type=jax.ShapeDtypeStruct((_D, _B), jnp.float32),
    mesh=_mesh,
    scratch_types=[
        pltpu.VMEM((_BPW,), jnp.int32),
        pltpu.VMEM((_EPW,), jnp.int32),
        pltpu.VMEM((_D, _BPW), jnp.float32),
        pltpu.SemaphoreType.DMA,
    ],
    compiler_params=pltpu.CompilerParams(
        use_tc_tiling_on_sc=False, needs_layout_passes=False),
)
def _sc_gather(tbl, idx, out, iv, ev, row, sem):
    wid = lax.axis_index("s") * _NC + lax.axis_index("c")
    row0 = wid * _BPW
    pltpu.sync_copy(idx.at[pl.ds(row0, _BPW)], iv)
    for j in range(_BPW // _L):
        v = iv[pl.ds(j * _L, _L)]
        for k in range(_D):
            ev[pl.ds(k * _BPW + j * _L, _L)] = v + (k * _V)
    copies = []
    for k in range(_D):
        isl = pl.ds(k * _BPW, _BPW)
        copies.append(pltpu.async_copy(
            tbl.at[ev.at[isl]], row.at[k], sem))
    for cp in copies:
        cp.wait()
    pltpu.sync_copy(row, out.at[:, pl.ds(row0, _BPW)])


@functools.partial(
    pl.kernel,
    out_type=(jax.ShapeDtypeStruct((_D, _B), jnp.float32),) * 2,
    mesh=_mesh,
    scratch_types=[
        pltpu.VMEM((_BPW,), jnp.int32),
        pltpu.VMEM((_BPW,), jnp.int32),
        pltpu.VMEM((_EPW,), jnp.int32),
        pltpu.VMEM((_EPW,), jnp.int32),
        pltpu.VMEM((_D, _BPW), jnp.float32),
        pltpu.VMEM((_D, _BPW), jnp.float32),
        pltpu.SemaphoreType.DMA,
    ],
    compiler_params=pltpu.CompilerParams(
        use_tc_tiling_on_sc=False, needs_layout_passes=False),
)
def _sc_gather2(f_tbl, o_tbl, f_idx, o_idx, out_f, out_o,
                fiv, oiv, fe, oe, frow, orow, sem):
    wid = lax.axis_index("s") * _NC + lax.axis_index("c")
    row0 = wid * _BPW
    pltpu.sync_copy(f_idx.at[pl.ds(row0, _BPW)], fiv)
    pltpu.sync_copy(o_idx.at[pl.ds(row0, _BPW)], oiv)
    copies = []
    for iv, ev, tbl, row in ((fiv, fe, f_tbl, frow), (oiv, oe, o_tbl, orow)):
        for j in range(_BPW // _L):
            v = iv[pl.ds(j * _L, _L)]
            for k in range(_D):
                ev[pl.ds(k * _BPW + j * _L, _L)] = v + (k * _V)
        for k in range(_D):
            isl = pl.ds(k * _BPW, _BPW)
            copies.append(pltpu.async_copy(
                tbl.at[ev.at[isl]], row.at[k], sem))
    for cp in copies:
        cp.wait()
    pltpu.sync_copy(frow, out_f.at[:, pl.ds(row0, _BPW)])
    pltpu.sync_copy(orow, out_o.at[:, pl.ds(row0, _BPW)])


_BLK = 2048


def _mlp_body(xc, xf, xo, xr, a_ref, b_ref, c_ref, x_ref,
              b1_ref, w2_ref, b2_ref, w3_ref, b3_ref, out):
    h = (jnp.dot(a_ref[...], xc[...], preferred_element_type=jnp.float32)
         + jnp.dot(b_ref[...], xf[...], preferred_element_type=jnp.float32)
         + jnp.dot(c_ref[...], xo[...], preferred_element_type=jnp.float32)
         + jnp.dot(x_ref[...], xr[...], preferred_element_type=jnp.float32)
         + b1_ref[...])
    h = jnp.maximum(h, 0.0)
    h = jnp.maximum(
        jnp.dot(w2_ref[...], h, preferred_element_type=jnp.float32)
        + b2_ref[...], 0.0)
    out[...] = (jnp.dot(w3_ref[...], h, preferred_element_type=jnp.float32)
                + b3_ref[...])


def _full(shape):
    return pl.BlockSpec(shape, lambda i: (0, 0))


_mlp_call = pl.pallas_call(
    _mlp_body,
    grid=(_B // _BLK,),
    in_specs=[
        pl.BlockSpec((_D, _BLK), lambda i: (0, i)),
        pl.BlockSpec((_D, _BLK), lambda i: (0, i)),
        pl.BlockSpec((_D, _BLK), lambda i: (0, i)),
        pl.BlockSpec((9, _BLK), lambda i: (0, i)),
        _full((32, _D)),
        _full((32, _D)),
        _full((32, _D)),
        _full((32, 9)),
        _full((32, 1)),
        _full((16, 32)),
        _full((16, 1)),
        _full((3, 16)),
        _full((3, 1)),
    ],
    out_specs=pl.BlockSpec((3, _BLK), lambda i: (0, i)),
    out_shape=jax.ShapeDtypeStruct((3, _B), jnp.float32),
)


def kernel(x, country_table, coffee_table, occupation_table,
           bn_weight, bn_bias, W1, b1, W2, b2, W3, b3):
    c_idx = x[:, 1].astype(jnp.int32)
    f_idx = x[:, 2].astype(jnp.int32)
    o_idx = x[:, 8].astype(jnp.int32)

    o_flat = occupation_table.T.reshape(-1)
    country_table, coffee_table, o_flat = lax.optimization_barrier(
        (country_table, coffee_table, o_flat))
    go = _sc_gather(o_flat, o_idx)
    gc, gf = _sc_gather2(country_table.T.reshape(-1),
                         coffee_table.T.reshape(-1), c_idx, f_idx)

    s = bn_weight * (1.0 / jnp.sqrt(1.0 + _EPS))
    w1s = W1 * s[:, None]
    b1p = (bn_bias @ W1 + b1).reshape(32, 1)
    wc_t = w1s[0:10].T
    wf_t = w1s[10:20].T
    wo_t = w1s[20:30].T
    wx_t = jnp.zeros((9, 32), jnp.float32).at[
        jnp.array([0, 3, 4, 5, 6, 7], dtype=jnp.int32)].set(w1s[30:36]).T

    out_t = _mlp_call(gc, gf, go, x.T,
                      wc_t, wf_t, wo_t, wx_t,
                      b1p, W2.T, b2.reshape(16, 1), W3.T, b3.reshape(3, 1))
    return out_t.T


# speedup vs baseline: 1.0941x; 1.0044x over previous
import functools

import jax
import jax.numpy as jnp
from jax import lax
from jax.experimental import pallas as pl
from jax.experimental.pallas import tpu as pltpu
from jax.experimental.pallas import tpu_sc as plsc

_B = 16384
_V = 100000
_D = 10
_EPS = 1e-5

_NC, _NS = 2, 16
_NW = _NC * _NS
_BPW = _B // _NW
_EPW = _BPW * _D
_CHUNK = 128
_L = 16

_mesh = plsc.VectorSubcoreMesh(core_axis_name="c", subcore_axis_name="s")


@functools.partial(
    pl.kernel,
    out_type=jax.ShapeDtypeStruct((_D, _B), jnp.float32),
    mesh=_mesh,
    scratch_types=[
        pltpu.VMEM((_BPW,), jnp.int32),
        pltpu.VMEM((_EPW,), jnp.int32),
        pltpu.VMEM((_D, _BPW), jnp.float32),
        pltpu.SemaphoreType.DMA,
        pltpu.SemaphoreType.DMA,
    ],
    compiler_params=pltpu.CompilerParams(
        use_tc_tiling_on_sc=False, needs_layout_passes=False),
)
def _sc_gather(tbl, idx, out, iv, ev, row, sem, osem):
    wid = lax.axis_index("s") * _NC + lax.axis_index("c")
    row0 = wid * _BPW
    pltpu.sync_copy(idx.at[pl.ds(row0, _BPW)], iv)
    for j in range(_BPW // _L):
        v = iv[pl.ds(j * _L, _L)]
        for k in range(_D):
            ev[pl.ds(k * _BPW + j * _L, _L)] = v + (k * _V)
    copies = []
    for k in range(_D):
        isl = pl.ds(k * _BPW, _BPW)
        copies.append(pltpu.async_copy(
            tbl.at[ev.at[isl]], row.at[k], sem))
    for cp in copies:
        cp.wait()
    pltpu.async_copy(row, out.at[:, pl.ds(row0, _BPW)], osem).wait()


@functools.partial(
    pl.kernel,
    out_type=(jax.ShapeDtypeStruct((_D, _B), jnp.float32),) * 2,
    mesh=_mesh,
    scratch_types=[
        pltpu.VMEM((_BPW,), jnp.int32),
        pltpu.VMEM((_BPW,), jnp.int32),
        pltpu.VMEM((_EPW,), jnp.int32),
        pltpu.VMEM((_EPW,), jnp.int32),
        pltpu.VMEM((_D, _BPW), jnp.float32),
        pltpu.VMEM((_D, _BPW), jnp.float32),
        pltpu.SemaphoreType.DMA,
        pltpu.SemaphoreType.DMA,
        pltpu.SemaphoreType.DMA,
    ],
    compiler_params=pltpu.CompilerParams(
        use_tc_tiling_on_sc=False, needs_layout_passes=False),
)
def _sc_gather2(f_tbl, o_tbl, f_idx, o_idx, out_f, out_o,
                fiv, oiv, fe, oe, frow, orow, fsem, osem2, osem):
    wid = lax.axis_index("s") * _NC + lax.axis_index("c")
    row0 = wid * _BPW
    pltpu.sync_copy(f_idx.at[pl.ds(row0, _BPW)], fiv)
    pltpu.sync_copy(o_idx.at[pl.ds(row0, _BPW)], oiv)
    copies = []
    for iv, ev, tbl, row, sem in ((fiv, fe, f_tbl, frow, fsem),
                                  (oiv, oe, o_tbl, orow, osem2)):
        for j in range(_BPW // _L):
            v = iv[pl.ds(j * _L, _L)]
            for k in range(_D):
                ev[pl.ds(k * _BPW + j * _L, _L)] = v + (k * _V)
        for k in range(_D):
            isl = pl.ds(k * _BPW, _BPW)
            copies.append(pltpu.async_copy(
                tbl.at[ev.at[isl]], row.at[k], sem))
    for cp in copies[:_D]:
        cp.wait()
    out1 = pltpu.async_copy(frow, out_f.at[:, pl.ds(row0, _BPW)], osem)
    for cp in copies[_D:]:
        cp.wait()
    out2 = pltpu.async_copy(orow, out_o.at[:, pl.ds(row0, _BPW)], osem)
    out1.wait()
    out2.wait()


_BLK = 2048


def _mlp_body(xc, xf, xo, xr, a_ref, b_ref, c_ref, x_ref,
              b1_ref, w2_ref, b2_ref, w3_ref, b3_ref, out):
    h = (jnp.dot(a_ref[...], xc[...], preferred_element_type=jnp.float32)
         + jnp.dot(b_ref[...], xf[...], preferred_element_type=jnp.float32)
         + jnp.dot(c_ref[...], xo[...], preferred_element_type=jnp.float32)
         + jnp.dot(x_ref[...], xr[...], preferred_element_type=jnp.float32)
         + b1_ref[...])
    h = jnp.maximum(h, 0.0)
    h = jnp.maximum(
        jnp.dot(w2_ref[...], h, preferred_element_type=jnp.float32)
        + b2_ref[...], 0.0)
    out[...] = (jnp.dot(w3_ref[...], h, preferred_element_type=jnp.float32)
                + b3_ref[...])


def _full(shape):
    return pl.BlockSpec(shape, lambda i: (0, 0))


_mlp_call = pl.pallas_call(
    _mlp_body,
    grid=(_B // _BLK,),
    in_specs=[
        pl.BlockSpec((_D, _BLK), lambda i: (0, i)),
        pl.BlockSpec((_D, _BLK), lambda i: (0, i)),
        pl.BlockSpec((_D, _BLK), lambda i: (0, i)),
        pl.BlockSpec((9, _BLK), lambda i: (0, i)),
        _full((32, _D)),
        _full((32, _D)),
        _full((32, _D)),
        _full((32, 9)),
        _full((32, 1)),
        _full((16, 32)),
        _full((16, 1)),
        _full((3, 16)),
        _full((3, 1)),
    ],
    out_specs=pl.BlockSpec((3, _BLK), lambda i: (0, i)),
    out_shape=jax.ShapeDtypeStruct((3, _B), jnp.float32),
)


def kernel(x, country_table, coffee_table, occupation_table,
           bn_weight, bn_bias, W1, b1, W2, b2, W3, b3):
    c_idx = x[:, 1].astype(jnp.int32)
    f_idx = x[:, 2].astype(jnp.int32)
    o_idx = x[:, 8].astype(jnp.int32)

    o_flat = occupation_table.T.reshape(-1)
    country_table, coffee_table, o_flat = lax.optimization_barrier(
        (country_table, coffee_table, o_flat))
    go = _sc_gather(o_flat, o_idx)
    gc, gf = _sc_gather2(country_table.T.reshape(-1),
                         coffee_table.T.reshape(-1), c_idx, f_idx)

    s = bn_weight * (1.0 / jnp.sqrt(1.0 + _EPS))
    w1s = W1 * s[:, None]
    b1p = (bn_bias @ W1 + b1).reshape(32, 1)
    wc_t = w1s[0:10].T
    wf_t = w1s[10:20].T
    wo_t = w1s[20:30].T
    wx_t = jnp.zeros((9, 32), jnp.float32).at[
        jnp.array([0, 3, 4, 5, 6, 7], dtype=jnp.int32)].set(w1s[30:36]).T

    out_t = _mlp_call(gc, gf, go, x.T,
                      wc_t, wf_t, wo_t, wx_t,
                      b1p, W2.T, b2.reshape(16, 1), W3.T, b3.reshape(3, 1))
    return out_t.T


# speedup vs baseline: 1.1250x; 1.0282x over previous
import functools

import jax
import jax.numpy as jnp
from jax import lax
from jax.experimental import pallas as pl
from jax.experimental.pallas import tpu as pltpu
from jax.experimental.pallas import tpu_sc as plsc

_B = 16384
_V = 100000
_D = 10
_EPS = 1e-5

_NC, _NS = 2, 16
_NW = _NC * _NS
_BPW = _B // _NW
_EPW = _BPW * _D
_CHUNK = 128
_L = 16

_mesh = plsc.VectorSubcoreMesh(core_axis_name="c", subcore_axis_name="s")


@functools.partial(
    pl.kernel,
    out_type=jax.ShapeDtypeStruct((_D, _B), jnp.float32),
    mesh=_mesh,
    scratch_types=[
        pltpu.VMEM((_BPW,), jnp.int32),
        pltpu.VMEM((_EPW,), jnp.int32),
        pltpu.VMEM((_D, _BPW), jnp.float32),
        pltpu.SemaphoreType.DMA,
        pltpu.SemaphoreType.DMA,
    ],
    compiler_params=pltpu.CompilerParams(
        use_tc_tiling_on_sc=False, needs_layout_passes=False),
)
def _sc_gather(tbl, idx, out, iv, ev, row, sem, osem):
    wid = lax.axis_index("s") * _NC + lax.axis_index("c")
    row0 = wid * _BPW
    pltpu.sync_copy(idx.at[pl.ds(row0, _BPW)], iv)
    for j in range(_BPW // _L):
        v = iv[pl.ds(j * _L, _L)]
        for k in range(_D):
            ev[pl.ds(k * _BPW + j * _L, _L)] = v + (k * _V)
    copies = []
    for k in range(_D):
        isl = pl.ds(k * _BPW, _BPW)
        copies.append(pltpu.async_copy(
            tbl.at[ev.at[isl]], row.at[k], sem))
    for cp in copies:
        cp.wait()
    pltpu.async_copy(row, out.at[:, pl.ds(row0, _BPW)], osem).wait()


@functools.partial(
    pl.kernel,
    out_type=jax.ShapeDtypeStruct((2 * _D, _B), jnp.float32),
    mesh=_mesh,
    scratch_types=[
        pltpu.VMEM((_BPW,), jnp.int32),
        pltpu.VMEM((_BPW,), jnp.int32),
        pltpu.VMEM((_EPW,), jnp.int32),
        pltpu.VMEM((_EPW,), jnp.int32),
        pltpu.VMEM((_D, _BPW), jnp.float32),
        pltpu.VMEM((_D, _BPW), jnp.float32),
        pltpu.SemaphoreType.DMA,
        pltpu.SemaphoreType.DMA,
        pltpu.SemaphoreType.DMA,
    ],
    compiler_params=pltpu.CompilerParams(
        use_tc_tiling_on_sc=False, needs_layout_passes=False),
)
def _sc_gather2(f_tbl, o_tbl, f_idx, o_idx, out_fo,
                fiv, oiv, fe, oe, frow, orow, fsem, osem2, osem):
    wid = lax.axis_index("s") * _NC + lax.axis_index("c")
    row0 = wid * _BPW
    pltpu.sync_copy(f_idx.at[pl.ds(row0, _BPW)], fiv)
    pltpu.sync_copy(o_idx.at[pl.ds(row0, _BPW)], oiv)
    copies = []
    for iv, ev, tbl, row, sem in ((fiv, fe, f_tbl, frow, fsem),
                                  (oiv, oe, o_tbl, orow, osem2)):
        for j in range(_BPW // _L):
            v = iv[pl.ds(j * _L, _L)]
            for k in range(_D):
                ev[pl.ds(k * _BPW + j * _L, _L)] = v + (k * _V)
        for k in range(_D):
            isl = pl.ds(k * _BPW, _BPW)
            copies.append(pltpu.async_copy(
                tbl.at[ev.at[isl]], row.at[k], sem))
    for cp in copies[:_D]:
        cp.wait()
    out1 = pltpu.async_copy(
        frow, out_fo.at[pl.ds(0, _D), pl.ds(row0, _BPW)], osem)
    for cp in copies[_D:]:
        cp.wait()
    out2 = pltpu.async_copy(
        orow, out_fo.at[pl.ds(_D, _D), pl.ds(row0, _BPW)], osem)
    out1.wait()
    out2.wait()


_BLK = 2048


def _mlp_body(xcf, xo, xr, a_ref, c_ref, x_ref,
              b1_ref, w2_ref, b2_ref, w3_ref, b3_ref, out):
    h = (jnp.dot(a_ref[...], xcf[...], preferred_element_type=jnp.float32)
         + jnp.dot(c_ref[...], xo[...], preferred_element_type=jnp.float32)
         + jnp.dot(x_ref[...], xr[...], preferred_element_type=jnp.float32)
         + b1_ref[...])
    h = jnp.maximum(h, 0.0)
    h = jnp.maximum(
        jnp.dot(w2_ref[...], h, preferred_element_type=jnp.float32)
        + b2_ref[...], 0.0)
    out[...] = (jnp.dot(w3_ref[...], h, preferred_element_type=jnp.float32)
                + b3_ref[...])


def _full(shape):
    return pl.BlockSpec(shape, lambda i: (0, 0))


_mlp_call = pl.pallas_call(
    _mlp_body,
    grid=(_B // _BLK,),
    in_specs=[
        pl.BlockSpec((2 * _D, _BLK), lambda i: (0, i)),
        pl.BlockSpec((_D, _BLK), lambda i: (0, i)),
        pl.BlockSpec((9, _BLK), lambda i: (0, i)),
        _full((32, 2 * _D)),
        _full((32, _D)),
        _full((32, 9)),
        _full((32, 1)),
        _full((16, 32)),
        _full((16, 1)),
        _full((3, 16)),
        _full((3, 1)),
    ],
    out_specs=pl.BlockSpec((3, _BLK), lambda i: (0, i)),
    out_shape=jax.ShapeDtypeStruct((3, _B), jnp.float32),
)


def kernel(x, country_table, coffee_table, occupation_table,
           bn_weight, bn_bias, W1, b1, W2, b2, W3, b3):
    o_idx = x[:, 8].astype(jnp.int32)
    o_flat = occupation_table.T.reshape(-1)
    country_table, coffee_table, x, o_flat = lax.optimization_barrier(
        (country_table, coffee_table, x, o_flat))
    go = _sc_gather(o_flat, o_idx)
    c_idx = x[:, 1].astype(jnp.int32)
    f_idx = x[:, 2].astype(jnp.int32)
    gcf = _sc_gather2(country_table.T.reshape(-1),
                      coffee_table.T.reshape(-1), c_idx, f_idx)

    s = bn_weight * (1.0 / jnp.sqrt(1.0 + _EPS))
    w1s = W1 * s[:, None]
    b1p = (bn_bias @ W1 + b1).reshape(32, 1)
    wcf_t = w1s[0:20].T
    wo_t = w1s[20:30].T
    wx_t = jnp.zeros((9, 32), jnp.float32).at[
        jnp.array([0, 3, 4, 5, 6, 7], dtype=jnp.int32)].set(w1s[30:36]).T

    out_t = _mlp_call(gcf, go, x.T,
                      wcf_t, wo_t, wx_t,
                      b1p, W2.T, b2.reshape(16, 1), W3.T, b3.reshape(3, 1))
    return out_t.T


# speedup vs baseline: 1.1253x; 1.0003x over previous
import functools

import jax
import jax.numpy as jnp
from jax import lax
from jax.experimental import pallas as pl
from jax.experimental.pallas import tpu as pltpu
from jax.experimental.pallas import tpu_sc as plsc

_B = 16384
_V = 100000
_D = 10
_EPS = 1e-5

_NC, _NS = 2, 16
_NW = _NC * _NS
_BPW = _B // _NW
_EPW = _BPW * _D
_L = 16

_mesh = plsc.VectorSubcoreMesh(core_axis_name="c", subcore_axis_name="s")


@functools.partial(
    pl.kernel,
    out_type=jax.ShapeDtypeStruct((_D, _B), jnp.float32),
    mesh=_mesh,
    scratch_types=[
        pltpu.VMEM((_BPW,), jnp.int32),
        pltpu.VMEM((_EPW,), jnp.int32),
        pltpu.VMEM((_D, _BPW), jnp.float32),
        pltpu.SemaphoreType.DMA,
        pltpu.SemaphoreType.DMA,
    ],
    compiler_params=pltpu.CompilerParams(
        use_tc_tiling_on_sc=False, needs_layout_passes=False),
)
def _sc_gather(tbl, idx, out, iv, ev, row, sem, osem):
    wid = lax.axis_index("s") * _NC + lax.axis_index("c")
    row0 = wid * _BPW
    pltpu.sync_copy(idx.at[pl.ds(row0, _BPW)], iv)
    for j in range(_BPW // _L):
        v = iv[pl.ds(j * _L, _L)]
        for k in range(_D):
            ev[pl.ds(k * _BPW + j * _L, _L)] = v + (k * _V)
    copies = []
    for k in range(_D):
        isl = pl.ds(k * _BPW, _BPW)
        copies.append(pltpu.async_copy(
            tbl.at[ev.at[isl]], row.at[k], sem))
    for cp in copies:
        cp.wait()
    pltpu.async_copy(row, out.at[:, pl.ds(row0, _BPW)], osem).wait()


@functools.partial(
    pl.kernel,
    out_type=jax.ShapeDtypeStruct((2 * _D, _B), jnp.float32),
    mesh=_mesh,
    scratch_types=[
        pltpu.VMEM((_BPW,), jnp.int32),
        pltpu.VMEM((_BPW,), jnp.int32),
        pltpu.VMEM((_EPW,), jnp.int32),
        pltpu.VMEM((_EPW,), jnp.int32),
        pltpu.VMEM((_D, _BPW), jnp.float32),
        pltpu.VMEM((_D, _BPW), jnp.float32),
        pltpu.SemaphoreType.DMA,
        pltpu.SemaphoreType.DMA,
        pltpu.SemaphoreType.DMA,
    ],
    compiler_params=pltpu.CompilerParams(
        use_tc_tiling_on_sc=False, needs_layout_passes=False),
)
def _sc_gather2(f_tbl, o_tbl, f_idx, o_idx, out_fo,
                fiv, oiv, fe, oe, frow, orow, fsem, osem2, osem):
    wid = lax.axis_index("s") * _NC + lax.axis_index("c")
    row0 = wid * _BPW
    pltpu.sync_copy(f_idx.at[pl.ds(row0, _BPW)], fiv)
    pltpu.sync_copy(o_idx.at[pl.ds(row0, _BPW)], oiv)
    copies = []
    for iv, ev, tbl, row, sem in ((fiv, fe, f_tbl, frow, fsem),
                                  (oiv, oe, o_tbl, orow, osem2)):
        for j in range(_BPW // _L):
            v = iv[pl.ds(j * _L, _L)]
            for k in range(_D):
                ev[pl.ds(k * _BPW + j * _L, _L)] = v + (k * _V)
        for k in range(_D):
            isl = pl.ds(k * _BPW, _BPW)
            copies.append(pltpu.async_copy(
                tbl.at[ev.at[isl]], row.at[k], sem))
    for cp in copies[:_D]:
        cp.wait()
    out1 = pltpu.async_copy(
        frow, out_fo.at[pl.ds(0, _D), pl.ds(row0, _BPW)], osem)
    for cp in copies[_D:]:
        cp.wait()
    out2 = pltpu.async_copy(
        orow, out_fo.at[pl.ds(_D, _D), pl.ds(row0, _BPW)], osem)
    out1.wait()
    out2.wait()


_BLK = 2048


def _mlp_body(xcf, xo, xr, a_ref, c_ref, x_ref,
              b1_ref, w2_ref, b2_ref, w3_ref, b3_ref, out):
    h = (jnp.dot(a_ref[...], xcf[...], preferred_element_type=jnp.float32)
         + jnp.dot(c_ref[...], xo[...], preferred_element_type=jnp.float32)
         + jnp.dot(x_ref[...], xr[...], preferred_element_type=jnp.float32)
         + b1_ref[...])
    h = jnp.maximum(h, 0.0)
    h = jnp.maximum(
        jnp.dot(w2_ref[...], h, preferred_element_type=jnp.float32)
        + b2_ref[...], 0.0)
    out[...] = (jnp.dot(w3_ref[...], h, preferred_element_type=jnp.float32)
                + b3_ref[...])


def _full(shape):
    return pl.BlockSpec(shape, lambda i: (0, 0))


_mlp_call = pl.pallas_call(
    _mlp_body,
    grid=(_B // _BLK,),
    in_specs=[
        pl.BlockSpec((2 * _D, _BLK), lambda i: (0, i)),
        pl.BlockSpec((_D, _BLK), lambda i: (0, i)),
        pl.BlockSpec((9, _BLK), lambda i: (0, i)),
        _full((32, 2 * _D)),
        _full((32, _D)),
        _full((32, 9)),
        _full((32, 1)),
        _full((16, 32)),
        _full((16, 1)),
        _full((3, 16)),
        _full((3, 1)),
    ],
    out_specs=pl.BlockSpec((3, _BLK), lambda i: (0, i)),
    out_shape=jax.ShapeDtypeStruct((3, _B), jnp.float32),
)


def kernel(x, country_table, coffee_table, occupation_table,
           bn_weight, bn_bias, W1, b1, W2, b2, W3, b3):
    o_idx = x[:, 8].astype(jnp.int32)
    o_flat = occupation_table.T.reshape(-1)
    country_table, coffee_table, x, o_flat = lax.optimization_barrier(
        (country_table, coffee_table, x, o_flat))
    go = _sc_gather(o_flat, o_idx)
    c_idx = x[:, 1].astype(jnp.int32)
    f_idx = x[:, 2].astype(jnp.int32)
    gcf = _sc_gather2(country_table.T.reshape(-1),
                      coffee_table.T.reshape(-1), c_idx, f_idx)

    s = bn_weight * (1.0 / jnp.sqrt(1.0 + _EPS))
    w1s = W1 * s[:, None]
    b1p = (bn_bias @ W1 + b1).reshape(32, 1)
    wcf_t = w1s[0:20].T
    wo_t = w1s[20:30].T
    wx_t = jnp.zeros((9, 32), jnp.float32).at[
        jnp.array([0, 3, 4, 5, 6, 7], dtype=jnp.int32)].set(w1s[30:36]).T

    out_t = _mlp_call(gcf, go, x.T,
                      wcf_t, wo_t, wx_t,
                      b1p, W2.T, b2.reshape(16, 1), W3.T, b3.reshape(3, 1))
    return out_t.T
